# Initial kernel scaffold; baseline (speedup 1.0000x reference)
import jax
import jax.numpy as jnp
from jax.experimental import pallas as pl


def kernel(commands, args, cmd_table, arg_table, W, b):
    raise NotImplementedError("write your pallas kernel here")


# speedup vs baseline: 6.2846x; 6.2846x over previous
import functools

import jax
import jax.numpy as jnp
from jax import lax
from jax.experimental import pallas as pl
from jax.experimental.pallas import tpu as pltpu
from jax.experimental.pallas import tpu_sc as plsc

_S, _N = 60, 4096
_SN = _S * _N
_NARGS = 16
_D = 256
_AE = 64
_NCMD = 6
_TBL_ROWS = _NCMD * 256 + (_NARGS - 1) * 256

_NC, _NS = 2, 16
_NW = _NC * _NS
_P = 128
_PER_W = _SN // _NW
_NBLK = _PER_W // _P
_NB_TOT = _SN // _P


def _build_table_body(at1_ref, w_ref, cmd_ref, b_ref, out_ref):
  at1 = at1_ref[...]
  t0 = jnp.dot(at1, w_ref[pl.ds(0, _AE), :],
               preferred_element_type=jnp.float32)
  t0 = t0 + b_ref[...]
  for c in range(_NCMD):
    out_ref[pl.ds(c * 256, 256), :] = t0 + cmd_ref[pl.ds(c, 1), :]
  for k in range(1, _NARGS):
    tk = jnp.dot(at1, w_ref[pl.ds(k * _AE, _AE), :],
                 preferred_element_type=jnp.float32)
    out_ref[pl.ds(_NCMD * 256 + (k - 1) * 256, 256), :] = tk


def _build_table(arg_table, W, cmd_table, b):
  at1 = arg_table[1:257]
  cmdp = jnp.pad(cmd_table, ((0, 2), (0, 0)))
  return pl.pallas_call(
      _build_table_body,
      out_shape=jax.ShapeDtypeStruct((_TBL_ROWS, _D), jnp.float32),
  )(at1, W, cmdp, b.reshape(1, _D))


def _accumulate(acc_v, st_v):
  def row(r, carry):
    for t in range(_D // 16):
      sl = pl.ds(t * 16, 16)
      plsc.addupdate(acc_v.at[r, sl], st_v[r, sl])
    return carry
  lax.fori_loop(0, _P, row, 0)


def _sc_body(slab_hbm, table_hbm, out_hbm, raw_v, idx_v, acc_v, st0_v, st1_v,
             sem_a, sem_0, sem_1):
  wid = lax.axis_index("s") * _NC + lax.axis_index("c")
  sts = (st0_v, st1_v)
  sems = (sem_0, sem_1)

  def block(j, carry):
    bidx = wid * _NBLK + j
    base = bidx * _P
    pltpu.sync_copy(slab_hbm.at[bidx], raw_v)
    for t in range(_P // 16):
      sl = pl.ds(t * 16, 16)
      idx_v[0, sl] = raw_v[0, sl] * 256 + raw_v[1, sl]
      for g in range(1, _NARGS):
        idx_v[g, sl] = raw_v[g + 1, sl] + (_NCMD * 256 + (g - 1) * 256)
    d_acc = pltpu.async_copy(table_hbm.at[idx_v.at[0]], acc_v, sem_a)
    d_cur = pltpu.async_copy(table_hbm.at[idx_v.at[1]], sts[1 % 2], sems[1 % 2])
    d_acc.wait()
    for g in range(1, _NARGS):
      if g < _NARGS - 1:
        d_nxt = pltpu.async_copy(
            table_hbm.at[idx_v.at[g + 1]], sts[(g + 1) % 2], sems[(g + 1) % 2])
      d_cur.wait()
      _accumulate(acc_v, sts[g % 2])
      if g < _NARGS - 1:
        d_cur = d_nxt
    pltpu.sync_copy(acc_v, out_hbm.at[pl.ds(base, _P)])
    return carry

  lax.fori_loop(0, _NBLK, block, 0)


def _sc_gather_sum(slab, table):
  mesh = plsc.VectorSubcoreMesh(core_axis_name="c", subcore_axis_name="s")
  f = pl.kernel(
      _sc_body,
      out_type=jax.ShapeDtypeStruct((_SN, _D), jnp.float32),
      mesh=mesh,
      scratch_types=[
          pltpu.VMEM((_NARGS + 1, _P), jnp.int32),
          pltpu.VMEM((_NARGS, _P), jnp.int32),
          pltpu.VMEM((_P, _D), jnp.float32),
          pltpu.VMEM((_P, _D), jnp.float32),
          pltpu.VMEM((_P, _D), jnp.float32),
          pltpu.SemaphoreType.DMA,
          pltpu.SemaphoreType.DMA,
          pltpu.SemaphoreType.DMA,
      ],
  )
  return f(slab, table)


def kernel(commands, args, cmd_table, arg_table, W, b):
  table = _build_table(arg_table, W, cmd_table, b)
  flat = jnp.concatenate(
      [commands.reshape(_SN, 1), args.reshape(_SN, _NARGS)], axis=1)
  slab = flat.reshape(_NB_TOT, _P, _NARGS + 1).swapaxes(1, 2)
  out = _sc_gather_sum(slab, table)
  return out.reshape(_S, _N, _D)
